# Initial kernel scaffold; baseline (speedup 1.0000x reference)
import jax
import jax.numpy as jnp
from jax.experimental import pallas as pl


def kernel(pred_logits, pred_boxes):
    raise NotImplementedError("write your pallas kernel here")


# speedup vs baseline: 1.1456x; 1.1456x over previous
import jax
import jax.numpy as jnp
from jax.experimental import pallas as pl

_NUM_CLASSES = 80
_K = 300


def _sigmoid_body(x_ref, o_ref):
    o_ref[...] = jax.nn.sigmoid(x_ref[...])


def kernel(pred_logits, pred_boxes):
    B, Q, C = pred_logits.shape
    flat = pred_logits.reshape(100000, 128)
    scores_flat = pl.pallas_call(
        _sigmoid_body,
        grid=(10,),
        in_specs=[pl.BlockSpec((10000, 128), lambda i: (i, 0))],
        out_specs=pl.BlockSpec((10000, 128), lambda i: (i, 0)),
        out_shape=jax.ShapeDtypeStruct((100000, 128), jnp.float32),
    )(flat)
    scores = scores_flat.reshape(B, Q * C)
    scores_top, index = jax.lax.top_k(scores, _K)
    labels = index - index // _NUM_CLASSES * _NUM_CLASSES
    qindex = index // _NUM_CLASSES
    boxes_top = jnp.take_along_axis(pred_boxes, qindex[:, :, None], axis=1)
    return scores_top, labels, boxes_top


# speedup vs baseline: 6.2633x; 5.4671x over previous
import functools
import jax
import jax.numpy as jnp
from jax import lax
from jax.experimental import pallas as pl
from jax.experimental.pallas import tpu as pltpu
from jax.experimental.pallas import tpu_sc as plsc

B = 8
Q = 20000
C = 80
K = 300
N = Q * C
NSUB = 32
SHARD = (B * N) // NSUB
NCH = 20
CH = SHARD // NCH
VREGS = CH // 16
CAP = 304
CAPB = CAP + 16

_mesh = plsc.VectorSubcoreMesh(core_axis_name="c", subcore_axis_name="s")
_cparams = pltpu.CompilerParams(needs_layout_passes=False)


def _wid():
    return lax.axis_index("s") * 2 + lax.axis_index("c")


def _iota16():
    return lax.iota(jnp.int32, 16)


def _sigmoid_body(x_ref, o_ref):
    o_ref[...] = jax.nn.sigmoid(x_ref[...])


def _sigmoid_tc(flat):
    return pl.pallas_call(
        _sigmoid_body,
        grid=(10,),
        in_specs=[pl.BlockSpec((10000, 128), lambda i: (i, 0))],
        out_specs=pl.BlockSpec((10000, 128), lambda i: (i, 0)),
        out_shape=jax.ShapeDtypeStruct((100000, 128), jnp.float32),
    )(flat)


def _zero_hist(hist, nwords):
    z = jnp.zeros((16,), jnp.int32)

    def zbody(i, _):
        hist[pl.ds(i * 16, 16)] = z
        return 0

    lax.fori_loop(0, nwords // 16, zbody, 0)


def _reduce_hist(hist, red, nb):
    def rbody(g, _):
        acc = hist[pl.ds(g * 16, 16)]
        for l in range(1, 16):
            acc = acc + hist[pl.ds(l * nb + g * 16, 16)]
        red[pl.ds(g * 16, 16)] = acc
        return 0

    lax.fori_loop(0, nb // 16, rbody, 0)


def _hist_kernel(level, nb, scores_hbm, cond_hbm, out_hbm, chunk, hist, red,
                 cond_v):
    w = _wid()
    base = w * SHARD
    _zero_hist(hist, 16 * nb)
    if level > 0:
        pltpu.sync_copy(cond_hbm.at[pl.ds(w * 16, 16)], cond_v)
    condv = cond_v[...] if level > 0 else None
    lanes = _iota16()
    ones = jnp.ones((16,), jnp.int32)

    def chunk_body(c, _):
        off = pl.multiple_of(base + c * CH, 8)
        pltpu.sync_copy(scores_hbm.at[pl.ds(off, CH)], chunk)

        def vbody(i, _):
            v = chunk[pl.ds(i * 16, 16)]
            bits = lax.bitcast_convert_type(v, jnp.int32)
            if level == 0:
                bucket = lax.shift_right_logical(bits, 21)
                mask = lanes == lanes
            elif level == 1:
                bucket = lax.shift_right_logical(bits, 10) & 0x7FF
                mask = lax.shift_right_logical(bits, 21) == condv
            else:
                bucket = bits & 0x3FF
                mask = lax.shift_right_logical(bits, 10) == condv
            idx = lanes * nb + bucket
            plsc.addupdate_scatter(hist, [idx], ones, mask=mask)
            return 0

        lax.fori_loop(0, VREGS, vbody, 0)
        return 0

    lax.fori_loop(0, NCH, chunk_body, 0)
    _reduce_hist(hist, red, nb)
    pltpu.sync_copy(red, out_hbm.at[pl.ds(w * nb, nb)])


def _make_hist(level, nb):
    kfn = functools.partial(_hist_kernel, level, nb)
    scratch = [
        pltpu.VMEM((CH,), jnp.float32),
        pltpu.VMEM((16 * nb,), jnp.int32),
        pltpu.VMEM((nb,), jnp.int32),
        pltpu.VMEM((16,), jnp.int32),
    ]
    return pl.kernel(
        kfn,
        out_type=jax.ShapeDtypeStruct((NSUB * nb,), jnp.int32),
        scratch_types=scratch,
        mesh=_mesh,
        compiler_params=_cparams,
    )


def _pick_bucket(hist, kvec):
    h = hist.reshape(B, 4, -1).sum(1)
    rc = jnp.cumsum(h[:, ::-1], axis=1)[:, ::-1]
    above = rc - h
    kc = kvec[:, None]
    cond = (above < kc) & (rc >= kc)
    bsel = jnp.argmax(cond, axis=1).astype(jnp.int32)
    gsel = jnp.take_along_axis(above, bsel[:, None], 1)[:, 0]
    return bsel, kvec - gsel


def _bcast32(x):
    return jnp.broadcast_to(jnp.repeat(x, 4)[:, None], (NSUB, 16))


def _compact_kernel(scores_hbm, v300_hbm, as_hbm, ai_hbm, ti_hbm, cnt_hbm,
                    chunk, as_v, ai_v, ti_v, v300_v, cnt_v):
    w = _wid()
    base = w * SHARD
    sbase = (w - (w // 4) * 4) * SHARD
    pltpu.sync_copy(v300_hbm.at[pl.ds(w * 16, 16)], v300_v)
    v300 = v300_v[...]
    lanes = _iota16()
    neg = jnp.full((16,), -1.0, jnp.float32)

    def init_body(i, _):
        as_v[pl.ds(i * 16, 16)] = neg
        return 0

    lax.fori_loop(0, CAPB // 16, init_body, 0)

    def chunk_body(c, offs):
        off = pl.multiple_of(base + c * CH, 8)
        pltpu.sync_copy(scores_hbm.at[pl.ds(off, CH)], chunk)

        def vbody(i, offs):
            og, ot = offs
            v = chunk[pl.ds(i * 16, 16)]
            bits = lax.bitcast_convert_type(v, jnp.int32)
            idxv = (sbase + c * CH + i * 16) + lanes
            m_g = bits > v300
            m_t = bits == v300
            mg_i = m_g.astype(jnp.int32)
            mt_i = m_t.astype(jnp.int32)
            pg = plsc.all_reduce_population_count(m_g)[0]
            pt = plsc.all_reduce_population_count(m_t)[0]
            posg = og + plsc.cumsum(mg_i) - mg_i
            post = ot + plsc.cumsum(mt_i) - mt_i
            plsc.store_scatter(as_v, [posg], v, mask=m_g)
            plsc.store_scatter(ai_v, [posg], idxv, mask=m_g)
            plsc.store_scatter(ti_v, [post], idxv, mask=m_t)
            og = jnp.minimum(og + pg, CAP)
            ot = jnp.minimum(ot + pt, CAP)
            return og, ot

        return lax.fori_loop(0, VREGS, vbody, offs)

    og, ot = lax.fori_loop(0, NCH, chunk_body,
                           (jnp.int32(0), jnp.int32(0)))
    cnt_v[...] = jnp.where(lanes == 0, og, jnp.where(lanes == 1, ot, 0))
    pltpu.sync_copy(as_v.at[pl.ds(0, CAP)], as_hbm.at[pl.ds(w * CAP, CAP)])
    pltpu.sync_copy(ai_v.at[pl.ds(0, CAP)], ai_hbm.at[pl.ds(w * CAP, CAP)])
    pltpu.sync_copy(ti_v.at[pl.ds(0, CAP)], ti_hbm.at[pl.ds(w * CAP, CAP)])
    pltpu.sync_copy(cnt_v, cnt_hbm.at[pl.ds(w * 16, 16)])


_compact = pl.kernel(
    _compact_kernel,
    out_type=(
        jax.ShapeDtypeStruct((NSUB * CAP,), jnp.float32),
        jax.ShapeDtypeStruct((NSUB * CAP,), jnp.int32),
        jax.ShapeDtypeStruct((NSUB * CAP,), jnp.int32),
        jax.ShapeDtypeStruct((NSUB * 16,), jnp.int32),
    ),
    scratch_types=[
        pltpu.VMEM((CH,), jnp.float32),
        pltpu.VMEM((CAPB,), jnp.float32),
        pltpu.VMEM((CAPB,), jnp.int32),
        pltpu.VMEM((CAPB,), jnp.int32),
        pltpu.VMEM((16,), jnp.int32),
        pltpu.VMEM((16,), jnp.int32),
    ],
    mesh=_mesh,
    compiler_params=_cparams,
)


WORK = 320


def _merge_kernel(as_hbm, ai_hbm, ti_hbm, cnt_hbm, v300f_hbm, boxes_hbm,
                  outs_hbm, outl_hbm, outb_hbm,
                  boxes_v, ws_v, wi_v, cs_v, ci_v, v300f_v, cnt_v,
                  st_s, st_l, st_q, st_b):
    w = _wid()
    lanes = _iota16()

    @pl.when(w < B)
    def _():
        b = w
        pltpu.sync_copy(boxes_hbm.at[pl.ds(b * Q * 4, Q * 4)], boxes_v)
        pltpu.sync_copy(v300f_hbm.at[pl.ds(b * 16, 16)], v300f_v)
        pltpu.sync_copy(cnt_hbm.at[pl.ds(b * 64, 64)], cnt_v)
        v300f = v300f_v[...]

        neg = jnp.full((16,), -1.0, jnp.float32)
        zero = jnp.zeros((16,), jnp.int32)

        def initw(i, _):
            ws_v[pl.ds(i * 16, 16)] = neg
            wi_v[pl.ds(i * 16, 16)] = zero
            return 0

        lax.fori_loop(0, WORK // 16, initw, 0)

        def initq(i, _):
            st_q[pl.ds(i * 16, 16)] = zero
            st_s[pl.ds(i * 16, 16)] = jnp.zeros((16,), jnp.float32)
            st_l[pl.ds(i * 16, 16)] = zero
            return 0

        lax.fori_loop(0, CAP // 16, initq, 0)

        off = jnp.int32(0)
        for s in range(4):
            na = cnt_v[pl.ds(s * 16, 16)][0]
            pltpu.sync_copy(as_hbm.at[pl.ds((4 * b + s) * CAP, CAP)], cs_v.at[pl.ds(0, CAP)])

            def cga(j, off):
                m = lanes < (na - j * 16)
                v = cs_v[pl.ds(j * 16, 16)]
                plsc.store_scatter(ws_v, [off + lanes], v, mask=m)
                return off + jnp.minimum(jnp.maximum(na - j * 16, 0), 16)

            off0 = off
            off = lax.fori_loop(0, CAP // 16, cga, off)
            pltpu.sync_copy(ai_hbm.at[pl.ds((4 * b + s) * CAP, CAP)], ci_v.at[pl.ds(0, CAP)])

            def cgi(j, off0):
                m = lanes < (na - j * 16)
                v = ci_v[pl.ds(j * 16, 16)]
                plsc.store_scatter(wi_v, [off0 + lanes], v, mask=m)
                return off0 + jnp.minimum(jnp.maximum(na - j * 16, 0), 16)

            lax.fori_loop(0, CAP // 16, cgi, off0)

        g = off
        trem = K - g
        for s in range(4):
            nt = jnp.minimum(cnt_v[pl.ds(s * 16, 16)][1], trem)
            pltpu.sync_copy(ti_hbm.at[pl.ds((4 * b + s) * CAP, CAP)], ci_v.at[pl.ds(0, CAP)])

            def ctg(j, off):
                m = lanes < (nt - j * 16)
                v = ci_v[pl.ds(j * 16, 16)]
                plsc.store_scatter(wi_v, [off + lanes], v, mask=m)
                plsc.store_scatter(ws_v, [off + lanes], v300f, mask=m)
                return off + jnp.minimum(jnp.maximum(nt - j * 16, 0), 16)

            off = lax.fori_loop(0, CAP // 16, ctg, off)
            trem = trem - nt

        def rank_body(e, _):
            ev = jnp.full((16,), e, jnp.int32)
            sev = plsc.load_gather(ws_v, [ev])
            iev = plsc.load_gather(wi_v, [ev])
            acc = jnp.zeros((16,), jnp.int32)

            def cmp_body(j, acc):
                fs = ws_v[pl.ds(j * 16, 16)]
                fi = wi_v[pl.ds(j * 16, 16)]
                m = (fs > sev) | ((fs == sev) & (fi < iev))
                return acc + plsc.all_reduce_population_count(m)

            acc = lax.fori_loop(0, WORK // 16, cmp_body, acc)
            m0 = lanes == 0
            qiv = iev // C
            plsc.store_scatter(st_s, [acc], sev, mask=m0)
            plsc.store_scatter(st_l, [acc], iev - qiv * C, mask=m0)
            plsc.store_scatter(st_q, [acc], qiv, mask=m0)
            return 0

        lax.fori_loop(0, K, rank_body, 0)

        def box_body(j, _):
            qv = st_q[pl.ds(j * 16, 16)]
            for cc in range(4):
                bx = plsc.load_gather(boxes_v, [qv * 4 + cc])
                bidx = (j * 16 + lanes) * 4 + cc
                plsc.store_scatter(st_b, [bidx], bx)
            return 0

        lax.fori_loop(0, CAP // 16, box_body, 0)

        pltpu.sync_copy(st_s, outs_hbm.at[pl.ds(b * CAP, CAP)])
        pltpu.sync_copy(st_l, outl_hbm.at[pl.ds(b * CAP, CAP)])
        pltpu.sync_copy(st_b, outb_hbm.at[pl.ds(b * CAP * 4, CAP * 4)])


_merge = pl.kernel(
    _merge_kernel,
    out_type=(
        jax.ShapeDtypeStruct((B * CAP,), jnp.float32),
        jax.ShapeDtypeStruct((B * CAP,), jnp.int32),
        jax.ShapeDtypeStruct((B * CAP * 4,), jnp.float32),
    ),
    scratch_types=[
        pltpu.VMEM((Q * 4,), jnp.float32),
        pltpu.VMEM((WORK,), jnp.float32),
        pltpu.VMEM((WORK,), jnp.int32),
        pltpu.VMEM((CAPB,), jnp.float32),
        pltpu.VMEM((CAPB,), jnp.int32),
        pltpu.VMEM((16,), jnp.float32),
        pltpu.VMEM((64,), jnp.int32),
        pltpu.VMEM((CAP,), jnp.float32),
        pltpu.VMEM((CAP,), jnp.int32),
        pltpu.VMEM((CAP,), jnp.int32),
        pltpu.VMEM((CAP * 4,), jnp.float32),
    ],
    mesh=_mesh,
    compiler_params=_cparams,
)

_hist1 = _make_hist(0, 512)
_hist2 = _make_hist(1, 2048)
_hist3 = _make_hist(2, 1024)


def kernel(pred_logits, pred_boxes):
    flat = pred_logits.reshape(100000, 128)
    scores = jax.nn.sigmoid(flat).reshape(B * N)
    dummy = jnp.zeros((NSUB * 16,), jnp.int32)

    h1 = _hist1(scores, dummy)
    need0 = jnp.full((B,), K, jnp.int32)
    b1, need1 = _pick_bucket(h1, need0)
    h2 = _hist2(scores, _bcast32(b1).reshape(-1))
    b2, need2 = _pick_bucket(h2, need1)
    h3 = _hist3(scores, _bcast32((b1 << 11) | b2).reshape(-1))
    b3, _ = _pick_bucket(h3, need2)
    v300 = (b1 << 21) | (b2 << 10) | b3

    cas, cai, cti, cnt = _compact(scores, _bcast32(v300).reshape(-1))
    v300f = jax.lax.bitcast_convert_type(v300, jnp.float32)
    v300f_b = jnp.broadcast_to(v300f[:, None], (B, 16)).reshape(-1)

    boxes_flat = pred_boxes.reshape(B * Q * 4)
    outs, outl, outb = _merge(cas, cai, cti, cnt, v300f_b, boxes_flat)

    scores_top = outs.reshape(B, CAP)[:, :K]
    labels = outl.reshape(B, CAP)[:, :K]
    boxes_top = outb.reshape(B, CAP * 4)[:, :K * 4].reshape(B, K, 4)
    return scores_top, labels, boxes_top
